# baseline (speedup 1.0000x reference)
import functools

import jax
import jax.numpy as jnp
from jax import lax
from jax.experimental import pallas as pl
from jax.experimental.pallas import tpu as pltpu
from jax.experimental.pallas import tpu_sc as plsc

_NC = 2
_NS = 16
_NW = _NC * _NS
_L = 16
_CK = 1024
_NB = 64


def _linear_body(relu, x_ref, w_ref, b_ref, o_ref):
    a = x_ref[...]
    if relu:
        a = jnp.maximum(a, 0.0)
    o_ref[...] = lax.dot_general(
        a, w_ref[...], (((1,), (1,)), ((), ())),
        preferred_element_type=jnp.float32) + b_ref[...]


def _linear(a, W, b, relu):
    M, Fin = a.shape
    Hout = W.shape[0]
    BM = 1000
    assert M % BM == 0
    return pl.pallas_call(
        functools.partial(_linear_body, relu),
        grid=(M // BM,),
        in_specs=[
            pl.BlockSpec((BM, Fin), lambda i: (i, 0)),
            pl.BlockSpec((Hout, Fin), lambda i: (0, 0)),
            pl.BlockSpec((1, Hout), lambda i: (0, 0)),
        ],
        out_specs=pl.BlockSpec((BM, Hout), lambda i: (i, 0)),
        out_shape=jax.ShapeDtypeStruct((M, Hout), jnp.float32),
    )(a, W, b.reshape(1, Hout))


def _make_scatter(n_nodes, hid, ep):
    rows = -(-n_nodes // (_NW * 8)) * 8
    out_rows = rows * _NW
    pend = _NB + _CK + _L
    pend = -(-pend // _L) * _L
    g_chunks = ep // _CK
    assert ep % (3 * _CK) == 0

    mesh = plsc.VectorSubcoreMesh(core_axis_name="c", subcore_axis_name="s")

    @functools.partial(
        pl.kernel,
        mesh=mesh,
        compiler_params=pltpu.CompilerParams(needs_layout_passes=False),
        out_type=jax.ShapeDtypeStruct((out_rows, hid), jnp.float32),
        scratch_types=[
            pltpu.VMEM((3 * _CK,), jnp.int32),
            pltpu.VMEM((3 * _CK,), jnp.int32),
            pltpu.VMEM((pend,), jnp.int32),
            pltpu.VMEM((pend,), jnp.int32),
            pltpu.VMEM((2, _NB), jnp.int32),
            pltpu.VMEM((2, _NB + _L), jnp.int32),
            pltpu.VMEM((2, _NB, hid), jnp.float32),
            pltpu.VMEM((rows, hid), jnp.float32),
            pltpu.SemaphoreType.DMA,
            pltpu.SemaphoreType.DMA,
            pltpu.SemaphoreType.DMA,
            pltpu.SemaphoreType.DMA,
            pltpu.SemaphoreType.DMA,
            pltpu.SemaphoreType.DMA,
            pltpu.SemaphoreType.DMA,
        ],
    )
    def scatter(h_hbm, src_hbm, dst_hbm, out_hbm,
                src_v, dst_v, psrc, ploc, gidx, gloc, rows_v, acc,
                sem, ss0, ss1, ss2, sd0, sd1, sd2):
        sem_s = [ss0, ss1, ss2]
        sem_d = [sd0, sd1, sd2]
        cid = lax.axis_index("c")
        sid = lax.axis_index("s")
        wid = sid * _NC + cid
        lo = wid * rows

        zeros_f = jnp.zeros((_L,), jnp.float32)

        def zrow(r, c):
            for j in range(hid // _L):
                acc[r, pl.ds(j * _L, _L)] = zeros_f
            return c
        lax.fori_loop(0, rows, zrow, 0)
        for j in range(pend // _L):
            psrc[pl.ds(j * _L, _L)] = jnp.zeros((_L,), jnp.int32)
            ploc[pl.ds(j * _L, _L)] = jnp.zeros((_L,), jnp.int32)

        def accumulate(pb, nblk, unroll):
            if nblk is _NB:
                def add_group(q, c):
                    r0 = q * _L
                    dls = gloc[pb, pl.ds(r0, _L)]
                    for k in range(_L):
                        dl = dls[k]
                        vals = [rows_v[pb, r0 + k, pl.ds(j * _L, _L)]
                                for j in range(hid // _L)]
                        for j in range(hid // _L):
                            plsc.addupdate(acc.at[dl, pl.ds(j * _L, _L)],
                                           vals[j])
                    return c
                lax.fori_loop(0, _NB // _L, add_group, 0, unroll=1)
            else:
                def add_row(r, c):
                    dl = gloc[pb, pl.ds(r, _L)][0]
                    for j in range(hid // _L):
                        sl = pl.ds(j * _L, _L)
                        plsc.addupdate(acc.at[dl, sl], rows_v[pb, r, sl])
                    return c
                lax.fori_loop(0, nblk, add_row, 0, unroll=unroll)

        def issue_gather(pb):
            pltpu.async_copy(h_hbm.at[gidx.at[pb]], rows_v.at[pb], sem)

        def wait_gather(pb):
            pltpu.make_async_copy(h_hbm.at[gidx.at[pb]], rows_v.at[pb],
                                  sem).wait()

        def issue(g, b):
            e0 = pl.multiple_of(g * _CK, 8)
            pltpu.async_copy(src_hbm.at[pl.ds(e0, _CK)],
                             src_v.at[pl.ds(b * _CK, _CK)], sem_s[b])
            pltpu.async_copy(dst_hbm.at[pl.ds(e0, _CK)],
                             dst_v.at[pl.ds(b * _CK, _CK)], sem_d[b])

        for b in range(3):
            issue(jnp.int32(b), b)

        def chunk(g, b, carry):
            pltpu.make_async_copy(src_hbm.at[pl.ds(0, _CK)],
                                  src_v.at[pl.ds(b * _CK, _CK)],
                                  sem_s[b]).wait()
            pltpu.make_async_copy(dst_hbm.at[pl.ds(0, _CK)],
                                  dst_v.at[pl.ds(b * _CK, _CK)],
                                  sem_d[b]).wait()

            @pl.when(g + 3 < g_chunks)
            def _():
                issue(g + 3, b)

            cnt, nblocks = carry
            cntv = jnp.full((_L,), cnt, jnp.int32)
            for i in range(_CK // _L):
                sl = pl.ds(b * _CK + i * _L, _L)
                s = src_v[sl]
                d = dst_v[sl]
                dl = d - lo
                mine = (d != s) & (dl.astype(jnp.uint32) < jnp.uint32(rows))
                mi = mine.astype(jnp.int32)
                cs = plsc.cumsum(mi)
                pos = cntv + cs - mi
                plsc.store_scatter(psrc, [pos], s, mask=mine)
                plsc.store_scatter(ploc, [pos], dl, mask=mine)
                cntv = cntv + plsc.all_reduce_population_count(mine)
            cnt = cntv[0]

            def flush(t):
                c, n = t
                pb = n % 2
                for k in range(_NB // _L):
                    ksl = pl.ds(k * _L, _L)
                    gidx[pb, ksl] = psrc[ksl]
                    gloc[pb, ksl] = ploc[ksl]

                @pl.when(n > 0)
                def _():
                    wait_gather(1 - pb)

                issue_gather(pb)

                @pl.when(n > 0)
                def _():
                    accumulate(1 - pb, _NB, unroll=4)

                for k in range((pend - _NB) // _L):
                    dst_sl = pl.ds(k * _L, _L)
                    src_sl = pl.ds(_NB + k * _L, _L)
                    psrc[dst_sl] = psrc[src_sl]
                    ploc[dst_sl] = ploc[src_sl]
                return c - _NB, n + 1

            return lax.while_loop(lambda t: t[0] >= _NB, flush, (cnt, nblocks))

        def triple(gg, carry):
            for b in range(3):
                carry = chunk(gg * 3 + b, b, carry)
            return carry

        cnt, nblocks = lax.fori_loop(
            0, g_chunks // 3, triple, (jnp.int32(0), jnp.int32(0)))

        @pl.when(nblocks > 0)
        def _():
            pb = (nblocks - 1) % 2
            wait_gather(pb)
            accumulate(pb, _NB, unroll=4)

        @pl.when(cnt > 0)
        def _():
            pb = nblocks % 2
            for k in range(_NB // _L):
                ksl = pl.ds(k * _L, _L)
                gidx[pb, ksl] = psrc[ksl]
                gloc[pb, ksl] = ploc[ksl]
            issue_gather(pb)
            wait_gather(pb)
            accumulate(pb, cnt, unroll=False)

        pltpu.sync_copy(acc,
                        out_hbm.at[pl.ds(pl.multiple_of(lo, 8), rows)])

    return scatter


def kernel(x, edge_index, W1, b1, W2, b2):
    n_nodes, _ = x.shape
    hid = W1.shape[0]
    e = edge_index.shape[1]

    h = _linear(x, W1, b1, relu=False)

    src = edge_index[0]
    dst = edge_index[1]
    ep = -(-e // (3 * _CK)) * (3 * _CK)
    if ep != e:
        pad = jnp.zeros((ep - e,), jnp.int32)
        src = jnp.concatenate([src, pad])
        dst = jnp.concatenate([dst, pad])

    agg = _make_scatter(n_nodes, hid, ep)(h, src, dst)[:n_nodes]

    return _linear(agg, W2, b2, relu=True)


# speedup vs baseline: 1.0055x; 1.0055x over previous
import functools

import jax
import jax.numpy as jnp
from jax import lax
from jax.experimental import pallas as pl
from jax.experimental.pallas import tpu as pltpu
from jax.experimental.pallas import tpu_sc as plsc

_NC = 2
_NS = 16
_NW = _NC * _NS
_L = 16
_CK = 1024
_NB = 64


def _linear_body(relu, x_ref, w_ref, b_ref, o_ref):
    a = x_ref[...]
    if relu:
        a = jnp.maximum(a, 0.0)
    o_ref[...] = lax.dot_general(
        a, w_ref[...], (((1,), (1,)), ((), ())),
        preferred_element_type=jnp.float32) + b_ref[...]


def _linear(a, W, b, relu):
    M, Fin = a.shape
    Hout = W.shape[0]
    BM = 1000
    assert M % BM == 0
    return pl.pallas_call(
        functools.partial(_linear_body, relu),
        grid=(M // BM,),
        in_specs=[
            pl.BlockSpec((BM, Fin), lambda i: (i, 0)),
            pl.BlockSpec((Hout, Fin), lambda i: (0, 0)),
            pl.BlockSpec((1, Hout), lambda i: (0, 0)),
        ],
        out_specs=pl.BlockSpec((BM, Hout), lambda i: (i, 0)),
        out_shape=jax.ShapeDtypeStruct((M, Hout), jnp.float32),
    )(a, W, b.reshape(1, Hout))


def _make_scatter(n_nodes, hid, ep):
    rows = -(-n_nodes // (_NW * 8)) * 8
    out_rows = rows * _NW
    pend = _NB + _CK + _L
    pend = -(-pend // _L) * _L
    g_chunks = ep // _CK
    assert ep % (3 * _CK) == 0

    mesh = plsc.VectorSubcoreMesh(core_axis_name="c", subcore_axis_name="s")

    @functools.partial(
        pl.kernel,
        mesh=mesh,
        compiler_params=pltpu.CompilerParams(needs_layout_passes=False),
        out_type=jax.ShapeDtypeStruct((out_rows, hid), jnp.float32),
        scratch_types=[
            pltpu.VMEM((3 * _CK,), jnp.int32),
            pltpu.VMEM((3 * _CK,), jnp.int32),
            pltpu.VMEM((pend,), jnp.int32),
            pltpu.VMEM((pend,), jnp.int32),
            pltpu.VMEM((2, _NB), jnp.int32),
            pltpu.VMEM((2, _NB + _L), jnp.int32),
            pltpu.VMEM((2, _NB, hid), jnp.float32),
            pltpu.VMEM((rows, hid), jnp.float32),
            pltpu.SemaphoreType.DMA,
            pltpu.SemaphoreType.DMA,
            pltpu.SemaphoreType.DMA,
            pltpu.SemaphoreType.DMA,
            pltpu.SemaphoreType.DMA,
            pltpu.SemaphoreType.DMA,
            pltpu.SemaphoreType.DMA,
        ],
    )
    def scatter(h_hbm, src_hbm, dst_hbm, out_hbm,
                src_v, dst_v, psrc, ploc, gidx, gloc, rows_v, acc,
                sem, ss0, ss1, ss2, sd0, sd1, sd2):
        sem_s = [ss0, ss1, ss2]
        sem_d = [sd0, sd1, sd2]
        cid = lax.axis_index("c")
        sid = lax.axis_index("s")
        wid = sid * _NC + cid
        lo = wid * rows

        zeros_f = jnp.zeros((_L,), jnp.float32)

        def zrow(r, c):
            for j in range(hid // _L):
                acc[r, pl.ds(j * _L, _L)] = zeros_f
            return c
        lax.fori_loop(0, rows, zrow, 0)
        for j in range(pend // _L):
            psrc[pl.ds(j * _L, _L)] = jnp.zeros((_L,), jnp.int32)
            ploc[pl.ds(j * _L, _L)] = jnp.zeros((_L,), jnp.int32)

        def accumulate(pb, nblk, unroll):
            if nblk is _NB:
                def add_group(q, c):
                    r0 = q * _L
                    dls = gloc[pb, pl.ds(r0, _L)]
                    for k in range(_L):
                        dl = dls[k]
                        vals = [rows_v[pb, r0 + k, pl.ds(j * _L, _L)]
                                for j in range(hid // _L)]
                        for j in range(hid // _L):
                            plsc.addupdate(acc.at[dl, pl.ds(j * _L, _L)],
                                           vals[j])
                    return c
                lax.fori_loop(0, _NB // _L, add_group, 0, unroll=1)
            else:
                def add_row(r, c):
                    dl = gloc[pb, pl.ds(r, _L)][0]
                    for j in range(hid // _L):
                        sl = pl.ds(j * _L, _L)
                        plsc.addupdate(acc.at[dl, sl], rows_v[pb, r, sl])
                    return c
                lax.fori_loop(0, nblk, add_row, 0, unroll=unroll)

        def issue_gather(pb):
            pltpu.async_copy(h_hbm.at[gidx.at[pb]], rows_v.at[pb], sem)

        def wait_gather(pb):
            pltpu.make_async_copy(h_hbm.at[gidx.at[pb]], rows_v.at[pb],
                                  sem).wait()

        def issue(g, b):
            e0 = pl.multiple_of(g * _CK, 8)
            pltpu.async_copy(src_hbm.at[pl.ds(e0, _CK)],
                             src_v.at[pl.ds(b * _CK, _CK)], sem_s[b])
            pltpu.async_copy(dst_hbm.at[pl.ds(e0, _CK)],
                             dst_v.at[pl.ds(b * _CK, _CK)], sem_d[b])

        for b in range(3):
            issue(jnp.int32(b), b)

        def chunk(g, b, carry):
            pltpu.make_async_copy(src_hbm.at[pl.ds(0, _CK)],
                                  src_v.at[pl.ds(b * _CK, _CK)],
                                  sem_s[b]).wait()
            pltpu.make_async_copy(dst_hbm.at[pl.ds(0, _CK)],
                                  dst_v.at[pl.ds(b * _CK, _CK)],
                                  sem_d[b]).wait()

            cnt, nblocks = carry
            cntv = jnp.full((_L,), cnt, jnp.int32)
            for i in range(_CK // _L):
                sl = pl.ds(b * _CK + i * _L, _L)
                s = src_v[sl]
                d = dst_v[sl]
                dl = d - lo
                mine = (d != s) & (dl.astype(jnp.uint32) < jnp.uint32(rows))
                mi = mine.astype(jnp.int32)
                cs = plsc.cumsum(mi)
                pos = cntv + cs - mi
                plsc.store_scatter(psrc, [pos], s, mask=mine)
                plsc.store_scatter(ploc, [pos], dl, mask=mine)
                cntv = cntv + plsc.all_reduce_population_count(mine)
            cnt = cntv[0]

            @pl.when(g + 3 < g_chunks)
            def _():
                issue(g + 3, b)

            def flush(t):
                c, n = t
                pb = n % 2
                for k in range(_NB // _L):
                    ksl = pl.ds(k * _L, _L)
                    gidx[pb, ksl] = psrc[ksl]
                    gloc[pb, ksl] = ploc[ksl]

                @pl.when(n > 0)
                def _():
                    wait_gather(1 - pb)

                issue_gather(pb)

                @pl.when(n > 0)
                def _():
                    accumulate(1 - pb, _NB, unroll=4)

                for k in range((pend - _NB) // _L):
                    dst_sl = pl.ds(k * _L, _L)
                    src_sl = pl.ds(_NB + k * _L, _L)
                    psrc[dst_sl] = psrc[src_sl]
                    ploc[dst_sl] = ploc[src_sl]
                return c - _NB, n + 1

            return lax.while_loop(lambda t: t[0] >= _NB, flush, (cnt, nblocks))

        def triple(gg, carry):
            for b in range(3):
                carry = chunk(gg * 3 + b, b, carry)
            return carry

        cnt, nblocks = lax.fori_loop(
            0, g_chunks // 3, triple, (jnp.int32(0), jnp.int32(0)))

        @pl.when(nblocks > 0)
        def _():
            pb = (nblocks - 1) % 2
            wait_gather(pb)
            accumulate(pb, _NB, unroll=4)

        @pl.when(cnt > 0)
        def _():
            pb = nblocks % 2
            for k in range(_NB // _L):
                ksl = pl.ds(k * _L, _L)
                gidx[pb, ksl] = psrc[ksl]
                gloc[pb, ksl] = ploc[ksl]
            issue_gather(pb)
            wait_gather(pb)
            accumulate(pb, cnt, unroll=False)

        pltpu.sync_copy(acc,
                        out_hbm.at[pl.ds(pl.multiple_of(lo, 8), rows)])

    return scatter


def kernel(x, edge_index, W1, b1, W2, b2):
    n_nodes, _ = x.shape
    hid = W1.shape[0]
    e = edge_index.shape[1]

    h = _linear(x, W1, b1, relu=False)

    src = edge_index[0]
    dst = edge_index[1]
    ep = -(-e // (3 * _CK)) * (3 * _CK)
    if ep != e:
        pad = jnp.zeros((ep - e,), jnp.int32)
        src = jnp.concatenate([src, pad])
        dst = jnp.concatenate([dst, pad])

    agg = _make_scatter(n_nodes, hid, ep)(h, src, dst)[:n_nodes]

    return _linear(agg, W2, b2, relu=True)


# speedup vs baseline: 1.5971x; 1.5884x over previous
import functools

import jax
import jax.numpy as jnp
from jax import lax
from jax.experimental import pallas as pl
from jax.experimental.pallas import tpu as pltpu
from jax.experimental.pallas import tpu_sc as plsc

_NC = 2
_NS = 16
_NW = _NC * _NS
_L = 16
_CK = 1024
_NB = 64


def _linear_body(relu, x_ref, w_ref, b_ref, o_ref):
    a = x_ref[...]
    if relu:
        a = jnp.maximum(a, 0.0)
    o_ref[...] = lax.dot_general(
        a, w_ref[...], (((1,), (1,)), ((), ())),
        preferred_element_type=jnp.float32) + b_ref[...]


def _linear(a, W, b, relu):
    M, Fin = a.shape
    Hout = W.shape[0]
    BM = 1000
    assert M % BM == 0
    return pl.pallas_call(
        functools.partial(_linear_body, relu),
        grid=(M // BM,),
        in_specs=[
            pl.BlockSpec((BM, Fin), lambda i: (i, 0)),
            pl.BlockSpec((Hout, Fin), lambda i: (0, 0)),
            pl.BlockSpec((1, Hout), lambda i: (0, 0)),
        ],
        out_specs=pl.BlockSpec((BM, Hout), lambda i: (i, 0)),
        out_shape=jax.ShapeDtypeStruct((M, Hout), jnp.float32),
    )(a, W, b.reshape(1, Hout))


def _make_scatter(n_nodes, hid, ep):
    rows = -(-n_nodes // (_NW * 8)) * 8
    out_rows = rows * _NW
    pend = _NB + _CK + _L
    pend = -(-pend // _L) * _L
    g_chunks = ep // _CK
    assert ep % _CK == 0

    mesh = plsc.VectorSubcoreMesh(core_axis_name="c", subcore_axis_name="s")

    @functools.partial(
        pl.kernel,
        mesh=mesh,
        compiler_params=pltpu.CompilerParams(needs_layout_passes=False),
        out_type=jax.ShapeDtypeStruct((out_rows, hid), jnp.float32),
        scratch_types=[
            pltpu.VMEM((2, _CK), jnp.int32),
            pltpu.VMEM((2, _CK), jnp.int32),
            pltpu.VMEM((pend,), jnp.int32),
            pltpu.VMEM((pend,), jnp.int32),
            pltpu.VMEM((2, _NB), jnp.int32),
            pltpu.VMEM((2, _NB + _L), jnp.int32),
            pltpu.VMEM((2, _NB, hid), jnp.float32),
            pltpu.VMEM((rows, hid), jnp.float32),
            pltpu.SemaphoreType.DMA,
            pltpu.SemaphoreType.DMA,
            pltpu.SemaphoreType.DMA,
        ],
    )
    def scatter(h_hbm, src_hbm, dst_hbm, out_hbm,
                src_v, dst_v, psrc, ploc, gidx, gloc, rows_v, acc,
                sem, sem_s, sem_d):
        cid = lax.axis_index("c")
        sid = lax.axis_index("s")
        wid = sid * _NC + cid
        lo = wid * rows

        zeros_f = jnp.zeros((_L,), jnp.float32)

        def zrow(r, c):
            for j in range(hid // _L):
                acc[r, pl.ds(j * _L, _L)] = zeros_f
            return c
        lax.fori_loop(0, rows, zrow, 0)
        for j in range(pend // _L):
            psrc[pl.ds(j * _L, _L)] = jnp.zeros((_L,), jnp.int32)
            ploc[pl.ds(j * _L, _L)] = jnp.zeros((_L,), jnp.int32)

        def accumulate(pb, nblk, unroll):
            if nblk is _NB:
                def add_group(q, c):
                    r0 = q * _L
                    dls = gloc[pb, pl.ds(r0, _L)]
                    for k in range(_L):
                        dl = dls[k]
                        vals = [rows_v[pb, r0 + k, pl.ds(j * _L, _L)]
                                for j in range(hid // _L)]
                        for j in range(hid // _L):
                            plsc.addupdate(acc.at[dl, pl.ds(j * _L, _L)],
                                           vals[j])
                    return c
                lax.fori_loop(0, _NB // _L, add_group, 0, unroll=1)
            else:
                def add_row(r, c):
                    dl = gloc[pb, pl.ds(r, _L)][0]
                    for j in range(hid // _L):
                        sl = pl.ds(j * _L, _L)
                        plsc.addupdate(acc.at[dl, sl], rows_v[pb, r, sl])
                    return c
                lax.fori_loop(0, nblk, add_row, 0, unroll=unroll)

        def issue_gather(pb):
            pltpu.async_copy(h_hbm.at[gidx.at[pb]], rows_v.at[pb], sem)

        def wait_gather(pb):
            pltpu.make_async_copy(h_hbm.at[gidx.at[pb]], rows_v.at[pb],
                                  sem).wait()

        def issue(g):
            e0 = pl.multiple_of(g * _CK, 8)
            b = g % 2
            pltpu.async_copy(src_hbm.at[pl.ds(e0, _CK)], src_v.at[b], sem_s)
            pltpu.async_copy(dst_hbm.at[pl.ds(e0, _CK)], dst_v.at[b], sem_d)

        issue(jnp.int32(0))

        def chunk(g, carry):
            b = g % 2
            pltpu.make_async_copy(src_hbm.at[pl.ds(0, _CK)], src_v.at[b],
                                  sem_s).wait()
            pltpu.make_async_copy(dst_hbm.at[pl.ds(0, _CK)], dst_v.at[b],
                                  sem_d).wait()

            @pl.when(g < g_chunks - 1)
            def _():
                issue(g + 1)

            cnt, nblocks = carry
            cntv = jnp.full((_L,), cnt, jnp.int32)
            for i in range(_CK // _L):
                sl = pl.ds(i * _L, _L)
                s = src_v[b, sl]
                d = dst_v[b, sl]
                dl = d - lo
                mine = (d != s) & (dl.astype(jnp.uint32) < jnp.uint32(rows))
                mi = mine.astype(jnp.int32)
                cs = plsc.cumsum(mi)
                pos = cntv + cs - mi
                plsc.store_scatter(psrc, [pos], s, mask=mine)
                plsc.store_scatter(ploc, [pos], dl, mask=mine)
                cntv = cntv + plsc.all_reduce_population_count(mine)
            cnt = cntv[0]

            def flush(t):
                c, n = t
                pb = n % 2
                for k in range(_NB // _L):
                    ksl = pl.ds(k * _L, _L)
                    gidx[pb, ksl] = psrc[ksl]
                    gloc[pb, ksl] = ploc[ksl]

                @pl.when(n > 0)
                def _():
                    wait_gather(1 - pb)

                issue_gather(pb)

                @pl.when(n > 0)
                def _():
                    accumulate(1 - pb, _NB, unroll=4)

                for k in range((pend - _NB) // _L):
                    dst_sl = pl.ds(k * _L, _L)
                    src_sl = pl.ds(_NB + k * _L, _L)
                    psrc[dst_sl] = psrc[src_sl]
                    ploc[dst_sl] = ploc[src_sl]
                return c - _NB, n + 1

            return lax.while_loop(lambda t: t[0] >= _NB, flush, (cnt, nblocks))

        cnt, nblocks = lax.fori_loop(
            0, g_chunks, chunk, (jnp.int32(0), jnp.int32(0)))

        @pl.when(nblocks > 0)
        def _():
            pb = (nblocks - 1) % 2
            wait_gather(pb)
            accumulate(pb, _NB, unroll=4)

        @pl.when(cnt > 0)
        def _():
            pb = nblocks % 2
            for k in range(_NB // _L):
                ksl = pl.ds(k * _L, _L)
                gidx[pb, ksl] = psrc[ksl]
                gloc[pb, ksl] = ploc[ksl]
            issue_gather(pb)
            wait_gather(pb)
            accumulate(pb, cnt, unroll=False)

        pltpu.sync_copy(acc,
                        out_hbm.at[pl.ds(pl.multiple_of(lo, 8), rows)])

    return scatter


def kernel(x, edge_index, W1, b1, W2, b2):
    n_nodes, _ = x.shape
    hid = W1.shape[0]
    e = edge_index.shape[1]

    h = _linear(x, W1, b1, relu=False)

    src = edge_index[0]
    dst = edge_index[1]
    ep = -(-e // _CK) * _CK
    if ep != e:
        pad = jnp.zeros((ep - e,), jnp.int32)
        src = jnp.concatenate([src, pad])
        dst = jnp.concatenate([dst, pad])

    agg = _make_scatter(n_nodes, hid, ep)(h, src, dst)[:n_nodes]

    return _linear(agg, W2, b2, relu=True)
